# baseline (speedup 1.0000x reference)
import jax
import jax.numpy as jnp
from jax.experimental import pallas as pl
from jax.experimental.pallas import tpu as pltpu

N = 4096
B = 4
IN_F = 256
OUT_F = 256

BM = 1024
BK = 1024


def _gcn_kernel(adj_ref, x_ref, w_ref, out_ref, acc_ref):
    k = pl.program_id(1)
    nk = pl.num_programs(1)

    @pl.when(k == 0)
    def _init():
        acc_ref[...] = jnp.zeros_like(acc_ref)

    acc_ref[...] += jnp.dot(
        adj_ref[...], x_ref[...], preferred_element_type=jnp.float32
    )

    @pl.when(k == nk - 1)
    def _finish():
        acc = acc_ref[...]
        w = w_ref[...]
        for b in range(B):
            blk = acc[:, b * IN_F : (b + 1) * IN_F]
            out_ref[:, b * OUT_F : (b + 1) * OUT_F] = jnp.dot(
                blk, w, preferred_element_type=jnp.float32
            )


@jax.jit
def kernel(input, adj, weight):
    x_r = jnp.reshape(input, (N, B * IN_F))
    grid = (N // BM, N // BK)
    out = pl.pallas_call(
        _gcn_kernel,
        grid=grid,
        in_specs=[
            pl.BlockSpec((BM, BK), lambda m, k: (m, k)),
            pl.BlockSpec((BK, B * IN_F), lambda m, k: (k, 0)),
            pl.BlockSpec((IN_F, OUT_F), lambda m, k: (0, 0)),
        ],
        out_specs=pl.BlockSpec((BM, B * OUT_F), lambda m, k: (m, 0)),
        out_shape=jax.ShapeDtypeStruct((N, B * OUT_F), jnp.float32),
        scratch_shapes=[pltpu.VMEM((BM, B * IN_F), jnp.float32)],
    )(adj, x_r, weight)
    return jnp.reshape(out, (N, B, OUT_F))


# speedup vs baseline: 1.0005x; 1.0005x over previous
import jax
import jax.numpy as jnp
from jax.experimental import pallas as pl
from jax.experimental.pallas import tpu as pltpu

N = 4096
B = 4
IN_F = 256
OUT_F = 256

BM = 1024
BK = 1024


def _gcn_kernel(adj_ref, x_ref, w_ref, out_ref, acc_ref):
    k = pl.program_id(1)
    nk = pl.num_programs(1)

    @pl.when(k == 0)
    def _init():
        acc_ref[...] = jnp.zeros_like(acc_ref)

    acc_ref[...] += jnp.dot(
        adj_ref[...].astype(jnp.bfloat16),
        x_ref[...].astype(jnp.bfloat16),
        preferred_element_type=jnp.float32,
    )

    @pl.when(k == nk - 1)
    def _finish():
        acc = acc_ref[...]
        w = w_ref[...].astype(jnp.bfloat16)
        for b in range(B):
            blk = acc[:, b * IN_F : (b + 1) * IN_F].astype(jnp.bfloat16)
            out_ref[:, b * OUT_F : (b + 1) * OUT_F] = jnp.dot(
                blk, w, preferred_element_type=jnp.float32
            )


@jax.jit
def kernel(input, adj, weight):
    x_r = jnp.reshape(input, (N, B * IN_F))
    grid = (N // BM, N // BK)
    out = pl.pallas_call(
        _gcn_kernel,
        grid=grid,
        in_specs=[
            pl.BlockSpec((BM, BK), lambda m, k: (m, k)),
            pl.BlockSpec((BK, B * IN_F), lambda m, k: (k, 0)),
            pl.BlockSpec((IN_F, OUT_F), lambda m, k: (0, 0)),
        ],
        out_specs=pl.BlockSpec((BM, B * OUT_F), lambda m, k: (m, 0)),
        out_shape=jax.ShapeDtypeStruct((N, B * OUT_F), jnp.float32),
        scratch_shapes=[pltpu.VMEM((BM, B * IN_F), jnp.float32)],
    )(adj, x_r, weight)
    return jnp.reshape(out, (N, B, OUT_F))
